# baseline (speedup 1.0000x reference)
import functools
import math

import jax
import jax.numpy as jnp
from jax.experimental import pallas as pl
from jax.experimental.pallas import tpu as pltpu
from jax.experimental.pallas import tpu_sc as plsc

D_MODEL = 64
SCALE = math.sqrt(D_MODEL)
WINDOW = 128
LANES = 16


def _embed(table, idx, n):
    mesh = plsc.VectorSubcoreMesh(core_axis_name="core", subcore_axis_name="subcore")

    @functools.partial(
        pl.kernel,
        out_type=jax.ShapeDtypeStruct((n, D_MODEL), table.dtype),
        mesh=mesh,
        compiler_params=pltpu.CompilerParams(use_tc_tiling_on_sc=False),
    )
    def run(table_hbm, idx_hbm, out_hbm):
        def body(i_vmem, o_vmem):
            pltpu.sync_copy(table_hbm.at[i_vmem.at[0]], o_vmem)

            @pl.loop(0, WINDOW)
            def _(r):
                for c in range(D_MODEL // LANES):
                    s = (pl.ds(r, 1), pl.ds(c * LANES, LANES))
                    o_vmem.at[s][...] = o_vmem.at[s][...] * SCALE

        pltpu.emit_pipeline(
            body,
            grid=(n // WINDOW,),
            in_specs=[pl.BlockSpec((1, WINDOW), index_map=lambda i: (0, i))],
            out_specs=[pl.BlockSpec((WINDOW, D_MODEL), index_map=lambda i: (i, 0))],
            core_axis_name=("core", "subcore"),
            dimension_semantics=(pltpu.PARALLEL,),
        )(idx_hbm, out_hbm)

    return run(table, idx)


def kernel(x, table):
    b, s = x.shape
    n = b * s
    idx = x.reshape(1, n).astype(jnp.int32)
    out = _embed(table, idx, n)
    return out.reshape(b, s, D_MODEL)


# speedup vs baseline: 1.4872x; 1.4872x over previous
import functools
import math

import jax
import jax.numpy as jnp
from jax import lax
from jax.experimental import pallas as pl
from jax.experimental.pallas import tpu as pltpu
from jax.experimental.pallas import tpu_sc as plsc

D_MODEL = 64
SCALE = math.sqrt(D_MODEL)
LANES = 16
W = 128
NB = 4
NW = 32


def _scale_block(src, dst):
    @pl.loop(0, W, step=4)
    def _(r0):
        for dr in range(4):
            for c in range(D_MODEL // LANES):
                s = (pl.ds(r0 + dr, 1), pl.ds(c * LANES, LANES))
                dst.at[s][...] = src.at[s][...] * SCALE


def _embed(table, idx, n):
    n_chunks = n // (NW * W)
    rounds = n_chunks // NB
    mesh = plsc.VectorSubcoreMesh(core_axis_name="core", subcore_axis_name="subcore")

    @functools.partial(
        pl.kernel,
        out_type=jax.ShapeDtypeStruct((n, D_MODEL), table.dtype),
        mesh=mesh,
        compiler_params=pltpu.CompilerParams(use_tc_tiling_on_sc=False),
        scratch_types=(
            [pltpu.VMEM((n_chunks, W), jnp.int32)]
            + [pltpu.VMEM((W, D_MODEL), jnp.float32) for _ in range(2 * NB)]
            + [pltpu.SemaphoreType.DMA for _ in range(2 * NB)]
        ),
    )
    def run(table_hbm, idx_hbm, out_hbm, idx_v, *bufs_and_sems):
        ibuf = bufs_and_sems[:NB]
        obuf = bufs_and_sems[NB:2 * NB]
        gsem = bufs_and_sems[2 * NB:3 * NB]
        wsem = bufs_and_sems[3 * NB:4 * NB]
        wid = lax.axis_index("core") * 16 + lax.axis_index("subcore")
        chunk0 = wid * n_chunks

        pltpu.sync_copy(idx_hbm.at[pl.ds(chunk0, n_chunks)], idx_v)

        def gather_start(b, c):
            pltpu.make_async_copy(
                table_hbm.at[idx_v.at[c]], ibuf[b], gsem[b]).start()

        def gather_wait(b, c):
            pltpu.make_async_copy(
                table_hbm.at[idx_v.at[c]], ibuf[b], gsem[b]).wait()

        def write_start(b, c):
            pltpu.make_async_copy(
                obuf[b], out_hbm.at[pl.ds((chunk0 + c) * W, W)], wsem[b]).start()

        def write_wait(b, c):
            pltpu.make_async_copy(
                obuf[b], out_hbm.at[pl.ds((chunk0 + c) * W, W)], wsem[b]).wait()

        for b in range(NB):
            gather_start(b, b)

        for b in range(NB):
            gather_wait(b, b)
            _scale_block(ibuf[b], obuf[b])
            gather_start(b, b + NB)
            write_start(b, b)

        @pl.loop(1, rounds)
        def _(r):
            c0 = r * NB
            for b in range(NB):
                c = c0 + b
                gather_wait(b, c)
                write_wait(b, c - NB)
                _scale_block(ibuf[b], obuf[b])

                @pl.when(c + NB < n_chunks)
                def _():
                    gather_start(b, c + NB)

                write_start(b, c)

        for b in range(NB):
            write_wait(b, n_chunks - NB + b)

    return run(table, idx)


def kernel(x, table):
    b, s = x.shape
    n = b * s
    idx = x.reshape(n // W, W).astype(jnp.int32)
    out = _embed(table, idx, n)
    return out.reshape(b, s, D_MODEL)
